# baseline (speedup 1.0000x reference)
import functools

import numpy as np

import jax
import jax.numpy as jnp
from jax import lax
from jax.experimental import pallas as pl
from jax.experimental.pallas import tpu as pltpu
from jax.experimental.pallas import tpu_sc as plsc

N_NODES = 10000
N_EDGES = 320000
D = 128

NC = 2
NS = 16
NW = NC * NS

GR = 128
NGC = 80
SUPER = 20
NSUP = NGC // SUPER
EW = GR * NGC
EPAD = NW * EW
ROWS_PER_TILE = 624
TAIL0 = NS * ROWS_PER_TILE
TAILN = N_NODES - TAIL0

_PERM = np.concatenate(
    [np.concatenate([32 * q + np.arange(0, 32, 2),
                     32 * q + np.arange(1, 32, 2)]) for q in range(4)])


def _sc_aggregate(sd, ew, xp):
    mesh = plsc.VectorSubcoreMesh(core_axis_name="c", subcore_axis_name="s")

    @functools.partial(
        pl.kernel,
        mesh=mesh,
        compiler_params=pltpu.CompilerParams(use_tc_tiling_on_sc=False),
        out_type=jax.ShapeDtypeStruct((NC, N_NODES, D), jnp.float32),
        scratch_types=[
            pltpu.VMEM_SHARED((N_NODES, D), jnp.float32),
            pltpu.VMEM((2, SUPER, 2, 128), jnp.int32),
            pltpu.VMEM((2, SUPER * GR + 16), jnp.float32),
            pltpu.VMEM((2, GR, D // 2), jnp.int32),
            pltpu.VMEM((GR, D), jnp.float32),
            pltpu.SemaphoreType.DMA,
            pltpu.SemaphoreType.DMA,
            pltpu.SemaphoreType.DMA,
            pltpu.SemaphoreType.DMA,
            pltpu.SemaphoreType.DMA,
        ],
    )
    def k(sd_hbm, ew_hbm, x_hbm, out_hbm, acc, sdb, ewb, gbuf, sbuf,
          gsem0, gsem1, ssem, bsem0, bsem1):
        gsem = (gsem0, gsem1)
        bsem = (bsem0, bsem1)
        c = lax.axis_index("c")
        sid = lax.axis_index("s")
        wid = c * NS + sid

        hb = {}

        def load_block(sblk):
            nb = sblk & 1
            return [
                pltpu.async_copy(
                    sd_hbm.at[pl.ds(wid * NGC + sblk * SUPER, SUPER)],
                    sdb.at[nb], bsem[nb]),
                pltpu.async_copy(
                    ew_hbm.at[pl.ds(wid * EW + sblk * SUPER * GR, SUPER * GR)],
                    ewb.at[nb, pl.ds(0, SUPER * GR)], bsem[nb]),
            ]

        hb[0] = load_block(0)

        zeros16 = jnp.zeros((16,), jnp.float32)

        def zrow(r, carry):
            for t in range(8):
                sbuf[r, pl.ds(t * 16, 16)] = zeros16
            return carry

        lax.fori_loop(0, GR, zrow, 0)
        row0 = pl.multiple_of(sid * ROWS_PER_TILE, 8)
        off = 0
        while off < ROWS_PER_TILE:
            sz = min(GR, ROWS_PER_TILE - off)
            pltpu.sync_copy(sbuf.at[pl.ds(0, sz)],
                            acc.at[pl.ds(row0 + off, sz)])
            off += sz

        @pl.when(sid == NS - 1)
        def _init_tail():
            pltpu.sync_copy(sbuf.at[pl.ds(0, TAILN)],
                            acc.at[pl.ds(TAIL0, TAILN)])

        plsc.subcore_barrier()

        for h in hb[0]:
            h.wait()
        hg = {}
        hs = {}

        def issue_gather(g, sblk2, ci2):
            bb = g & 1
            return pltpu.async_copy(
                x_hbm.at[sdb.at[sblk2 & 1, ci2, 0]], gbuf.at[bb], gsem[bb])

        hg[0] = issue_gather(0, 0, 0)

        def make_scale(b, bs, ci):
            def scale_edge(e):
                wsh = ewb[bs, pl.ds(ci * GR + e, 16)]
                wb = jnp.full((16,), wsh[0], jnp.float32)
                packed = [gbuf[b, e, pl.ds(q * 16, 16)] for q in range(4)]
                for q in range(4):
                    lo = lax.bitcast_convert_type(
                        lax.shift_left(packed[q], 16), jnp.float32)
                    hi = lax.bitcast_convert_type(
                        jnp.bitwise_and(packed[q], jnp.int32(-65536)),
                        jnp.float32)
                    sbuf[e, pl.ds(q * 32, 16)] = lo * wb
                    sbuf[e, pl.ds(q * 32 + 16, 16)] = hi * wb
            return scale_edge

        for g in range(NGC):
            b = g & 1
            sblk = g // SUPER
            bs = sblk & 1
            ci = g % SUPER
            if ci == 0 and sblk + 1 < NSUP:
                hb[sblk + 1] = load_block(sblk + 1)
            hg[g].wait()
            if g + 1 < NGC:
                nsblk = (g + 1) // SUPER
                nci = (g + 1) % SUPER
                if nci == 0:
                    for h in hb[nsblk]:
                        h.wait()
                hg[g + 1] = issue_gather(g + 1, nsblk, nci)
            if g >= 1:
                hs[g - 1].wait()
            plsc.parallel_loop(0, GR, 1, unroll=2)(make_scale(b, bs, ci))
            hs[g] = pltpu.async_copy(sbuf, acc.at[sdb.at[bs, ci, 1]],
                                     ssem, add=True)

        hs[NGC - 1].wait()
        plsc.subcore_barrier()

        pltpu.sync_copy(acc.at[pl.ds(row0, ROWS_PER_TILE)],
                        out_hbm.at[c, pl.ds(row0, ROWS_PER_TILE)])

        @pl.when(sid == NS - 1)
        def _pub_tail():
            pltpu.sync_copy(acc.at[pl.ds(TAIL0, TAILN)],
                            out_hbm.at[c, pl.ds(TAIL0, TAILN)])

    return k(sd, ew, xp)


BM = 1000


def _tc_body(p_ref, w_ref, b_ref, o_ref):
    agg = p_ref[0] + p_ref[1]
    o_ref[...] = jnp.dot(agg, w_ref[...],
                         preferred_element_type=jnp.float32,
                         precision=lax.Precision.HIGHEST) + b_ref[...]


_tc_final = pl.pallas_call(
    _tc_body,
    grid=(N_NODES // BM,),
    in_specs=[
        pl.BlockSpec((NC, BM, D), lambda i: (0, i, 0)),
        pl.BlockSpec((D, D), lambda i: (0, 0)),
        pl.BlockSpec((1, D), lambda i: (0, 0)),
    ],
    out_specs=pl.BlockSpec((BM, D), lambda i: (i, 0)),
    out_shape=jax.ShapeDtypeStruct((N_NODES, D), jnp.float32),
)


def kernel(x, edge_index, edge_weight, weight, bias):
    dst = edge_index[0].astype(jnp.int32)
    src = edge_index[1].astype(jnp.int32)
    pad = EPAD - N_EDGES
    src2d = jnp.concatenate([src, jnp.zeros((pad,), jnp.int32)]).reshape(
        EPAD // 128, 128)
    dst2d = jnp.concatenate([dst, jnp.zeros((pad,), jnp.int32)]).reshape(
        EPAD // 128, 128)
    sd = jnp.stack([src2d, dst2d], axis=1)
    ew = jnp.concatenate([edge_weight.astype(jnp.float32),
                          jnp.zeros((pad,), jnp.float32)])
    xp = jax.lax.bitcast_convert_type(
        x.astype(jnp.bfloat16).reshape(N_NODES, D // 2, 2),
        jnp.int32)
    partials = _sc_aggregate(sd, ew, xp)
    w_perm = weight[jnp.asarray(_PERM)]
    return _tc_final(partials, w_perm, bias.reshape(1, D))


# speedup vs baseline: 1.0556x; 1.0556x over previous
import functools

import numpy as np

import jax
import jax.numpy as jnp
from jax import lax
from jax.experimental import pallas as pl
from jax.experimental.pallas import tpu as pltpu
from jax.experimental.pallas import tpu_sc as plsc

N_NODES = 10000
N_EDGES = 320000
D = 128

NC = 2
NS = 16
NW = NC * NS

GR = 128
NGC = 80
SUPER = 10
NSUP = NGC // SUPER
EW = GR * NGC
EPAD = NW * EW
ROWS_PER_TILE = 624
TAIL0 = NS * ROWS_PER_TILE
TAILN = N_NODES - TAIL0

_PERM = np.concatenate(
    [np.concatenate([32 * q + np.arange(0, 32, 2),
                     32 * q + np.arange(1, 32, 2)]) for q in range(4)])


def _sc_aggregate(sd, ew, xp):
    mesh = plsc.VectorSubcoreMesh(core_axis_name="c", subcore_axis_name="s")

    @functools.partial(
        pl.kernel,
        mesh=mesh,
        compiler_params=pltpu.CompilerParams(use_tc_tiling_on_sc=False),
        out_type=jax.ShapeDtypeStruct((NC, N_NODES, D), jnp.float32),
        scratch_types=[
            pltpu.VMEM_SHARED((N_NODES, D), jnp.float32),
            pltpu.VMEM((2, SUPER, 2, 128), jnp.int32),
            pltpu.VMEM((2, SUPER * GR + 16), jnp.float32),
            pltpu.VMEM((3, GR, D // 2), jnp.int32),
            pltpu.VMEM((GR, D), jnp.float32),
            pltpu.SemaphoreType.DMA,
            pltpu.SemaphoreType.DMA,
            pltpu.SemaphoreType.DMA,
            pltpu.SemaphoreType.DMA,
            pltpu.SemaphoreType.DMA,
            pltpu.SemaphoreType.DMA,
        ],
    )
    def k(sd_hbm, ew_hbm, x_hbm, out_hbm, acc, sdb, ewb, gbuf, sbuf,
          gsem0, gsem1, gsem2, ssem, bsem0, bsem1):
        gsem = (gsem0, gsem1, gsem2)
        bsem = (bsem0, bsem1)
        c = lax.axis_index("c")
        sid = lax.axis_index("s")
        wid = c * NS + sid

        hb = {}

        def load_block(sblk):
            nb = sblk & 1
            return [
                pltpu.async_copy(
                    sd_hbm.at[pl.ds(wid * NGC + sblk * SUPER, SUPER)],
                    sdb.at[nb], bsem[nb]),
                pltpu.async_copy(
                    ew_hbm.at[pl.ds(wid * EW + sblk * SUPER * GR, SUPER * GR)],
                    ewb.at[nb, pl.ds(0, SUPER * GR)], bsem[nb]),
            ]

        hb[0] = load_block(0)

        zeros16 = jnp.zeros((16,), jnp.float32)

        def zrow(r, carry):
            for t in range(8):
                sbuf[r, pl.ds(t * 16, 16)] = zeros16
            return carry

        lax.fori_loop(0, GR, zrow, 0)
        row0 = pl.multiple_of(sid * ROWS_PER_TILE, 8)
        off = 0
        while off < ROWS_PER_TILE:
            sz = min(GR, ROWS_PER_TILE - off)
            pltpu.sync_copy(sbuf.at[pl.ds(0, sz)],
                            acc.at[pl.ds(row0 + off, sz)])
            off += sz

        @pl.when(sid == NS - 1)
        def _init_tail():
            pltpu.sync_copy(sbuf.at[pl.ds(0, TAILN)],
                            acc.at[pl.ds(TAIL0, TAILN)])

        plsc.subcore_barrier()

        for h in hb[0]:
            h.wait()
        hg = {}
        hs = {}

        def issue_gather(g, sblk2, ci2):
            bb = g % 3
            return pltpu.async_copy(
                x_hbm.at[sdb.at[sblk2 & 1, ci2, 0]], gbuf.at[bb], gsem[bb])

        hg[0] = issue_gather(0, 0, 0)
        hg[1] = issue_gather(1, 0, 1)

        def make_scale(bq, bs, ci):
            def scale_edge(e):
                wsh = ewb[bs, pl.ds(ci * GR + e, 16)]
                wb = jnp.full((16,), wsh[0], jnp.float32)
                packed = [gbuf[bq, e, pl.ds(q * 16, 16)] for q in range(4)]
                for q in range(4):
                    lo = lax.bitcast_convert_type(
                        lax.shift_left(packed[q], 16), jnp.float32)
                    hi = lax.bitcast_convert_type(
                        jnp.bitwise_and(packed[q], jnp.int32(-65536)),
                        jnp.float32)
                    sbuf[e, pl.ds(q * 32, 16)] = lo * wb
                    sbuf[e, pl.ds(q * 32 + 16, 16)] = hi * wb
            return scale_edge

        for g in range(NGC):
            b = g % 3
            sblk = g // SUPER
            bs = sblk & 1
            ci = g % SUPER
            hg[g].wait()
            if g >= 1:
                hs[g - 1].wait()
            if ci == 0 and sblk + 1 < NSUP:
                hb[sblk + 1] = load_block(sblk + 1)
            if g + 2 < NGC:
                nsblk = (g + 2) // SUPER
                nci = (g + 2) % SUPER
                if nci == 0:
                    for h in hb[nsblk]:
                        h.wait()
                hg[g + 2] = issue_gather(g + 2, nsblk, nci)
            plsc.parallel_loop(0, GR, 1, unroll=2)(make_scale(b, bs, ci))
            hs[g] = pltpu.async_copy(sbuf, acc.at[sdb.at[bs, ci, 1]],
                                     ssem, add=True)

        hs[NGC - 1].wait()
        plsc.subcore_barrier()

        pltpu.sync_copy(acc.at[pl.ds(row0, ROWS_PER_TILE)],
                        out_hbm.at[c, pl.ds(row0, ROWS_PER_TILE)])

        @pl.when(sid == NS - 1)
        def _pub_tail():
            pltpu.sync_copy(acc.at[pl.ds(TAIL0, TAILN)],
                            out_hbm.at[c, pl.ds(TAIL0, TAILN)])

    return k(sd, ew, xp)


BM = 1000


def _tc_body(p_ref, w_ref, b_ref, o_ref):
    agg = p_ref[0] + p_ref[1]
    o_ref[...] = jnp.dot(agg, w_ref[...],
                         preferred_element_type=jnp.float32,
                         precision=lax.Precision.HIGHEST) + b_ref[...]


_tc_final = pl.pallas_call(
    _tc_body,
    grid=(N_NODES // BM,),
    in_specs=[
        pl.BlockSpec((NC, BM, D), lambda i: (0, i, 0)),
        pl.BlockSpec((D, D), lambda i: (0, 0)),
        pl.BlockSpec((1, D), lambda i: (0, 0)),
    ],
    out_specs=pl.BlockSpec((BM, D), lambda i: (i, 0)),
    out_shape=jax.ShapeDtypeStruct((N_NODES, D), jnp.float32),
)


def kernel(x, edge_index, edge_weight, weight, bias):
    dst = edge_index[0].astype(jnp.int32)
    src = edge_index[1].astype(jnp.int32)
    pad = EPAD - N_EDGES
    src2d = jnp.concatenate([src, jnp.zeros((pad,), jnp.int32)]).reshape(
        EPAD // 128, 128)
    dst2d = jnp.concatenate([dst, jnp.zeros((pad,), jnp.int32)]).reshape(
        EPAD // 128, 128)
    sd = jnp.stack([src2d, dst2d], axis=1)
    ew = jnp.concatenate([edge_weight.astype(jnp.float32),
                          jnp.zeros((pad,), jnp.float32)])
    xp = jax.lax.bitcast_convert_type(
        x.astype(jnp.bfloat16).reshape(N_NODES, D // 2, 2),
        jnp.int32)
    partials = _sc_aggregate(sd, ew, xp)
    w_perm = weight[jnp.asarray(_PERM)]
    return _tc_final(partials, w_perm, bias.reshape(1, D))
